# baseline (speedup 1.0000x reference)
import functools
import jax
import jax.numpy as jnp
from jax import lax
from jax.experimental import pallas as pl
from jax.experimental.pallas import tpu as pltpu
from jax.experimental.pallas import tpu_sc as plsc

_NG = 32
_BE = 4096


def _mm(a, b):
    return jax.lax.dot_general(a, b, (((1,), (0,)), ((), ())),
                               preferred_element_type=jnp.float32)


def _sig(v):
    return 1.0 / (1.0 + jnp.exp(-v))


def _node_enc_kernel(nodes_ref, aa_ref, ss_ref, w1_ref, b1_ref, w2_ref,
                     b2_ref, ea_ref, es_ref, wex_ref, be_ref,
                     xh_ref, p_ref):
    h = jnp.maximum(_mm(nodes_ref[...], w1_ref[...]) + b1_ref[...], 0.0)
    h = jnp.maximum(_mm(h, w2_ref[...]) + b2_ref[...], 0.0)
    oh_aa = (aa_ref[...] == jax.lax.broadcasted_iota(jnp.int32, (1, 24), 1)
             ).astype(jnp.float32)
    oh_ss = (ss_ref[...] == jax.lax.broadcasted_iota(jnp.int32, (1, 16), 1)
             ).astype(jnp.float32)
    xh = jnp.concatenate(
        [_mm(oh_aa, ea_ref[...]), h, _mm(oh_ss, es_ref[...])], axis=1)
    xh_ref[...] = xh
    p_ref[...] = _mm(xh, wex_ref[...]) + be_ref[...]


def _node_encoder(nodes, aa2, ss2, p):
    N = nodes.shape[0]
    emb_aa = jnp.zeros((24, 64), jnp.float32).at[:20].set(p['emb_aa'])
    emb_ss = jnp.zeros((16, 16), jnp.float32).at[:9].set(p['emb_ss'])
    wex = p['mp'][0]['edge_w'][:208]
    be = p['mp'][0]['edge_b'][None, :]
    return pl.pallas_call(
        _node_enc_kernel,
        out_shape=(jax.ShapeDtypeStruct((N, 208), jnp.float32),
                   jax.ShapeDtypeStruct((N, 128), jnp.float32)),
    )(nodes, aa2, ss2, p['enc_node_w1'], p['enc_node_b1'][None],
      p['enc_node_w2'], p['enc_node_b2'][None], emb_aa, emb_ss, wex, be)


def _eg_accum(i, bsrc_ref, e_new, eg_ref):
    ohbt = (bsrc_ref[0] == jax.lax.broadcasted_iota(jnp.int32, (40, 1), 0)
            ).astype(jnp.float32)
    ones = jnp.ones((e_new.shape[0], 8), jnp.float32)
    egp = _mm(ohbt, jnp.concatenate([e_new, ones], axis=1))

    @pl.when(i == 0)
    def _():
        eg_ref[...] = jnp.zeros_like(eg_ref)

    eg_ref[...] += egp


def _edge1_kernel(ea_ref, src_ref, dst_ref, bsrc_ref, g_ref, ew1_ref,
                  eb1_ref, ew2_ref, eb2_ref, tsep_ref, wenc_ref, wrbf_ref,
                  out_ref, eg_ref):
    ea = ea_ref[...]
    a = jnp.maximum(_mm(ea, ew1_ref[...]) + eb1_ref[...], 0.0)
    b = jnp.maximum(_mm(a, ew2_ref[...]) + eb2_ref[...], 0.0)
    acc = _mm(b, wenc_ref[...])
    d = jnp.abs(src_ref[...] - dst_ref[...])
    code = ((d > 1).astype(jnp.int32) + (d > 2) + (d > 3) + (d > 4)
            + (d > 5) + (d > 10) + (d > 15))
    oh = (code == jax.lax.broadcasted_iota(jnp.int32, (1, 8), 1)
          ).astype(jnp.float32)
    acc = acc + _mm(oh, tsep_ref[...])
    cen = jax.lax.broadcasted_iota(jnp.int32, (1, 16), 1).astype(
        jnp.float32) * (20.0 / 15.0)
    rbf = jnp.exp(-(ea[:, 0:1] - cen) ** 2)
    acc = acc + _mm(rbf, wrbf_ref[...])
    e_new = jnp.maximum(acc + g_ref[...], 0.0)
    out_ref[...] = e_new
    _eg_accum(pl.program_id(0), bsrc_ref, e_new, eg_ref)


def _edge1(ea, src2, dst2, bsrc3, g, p):
    EP = ea.shape[0]
    mp = p['mp'][0]
    wenc = mp['edge_w'][208:272]
    tsep = p['emb_sep'] @ mp['edge_w'][272:304]
    wrbf = mp['edge_w'][304:320]
    rows = lambda d: pl.BlockSpec((_BE, d), lambda i: (i, 0))
    full = lambda s: pl.BlockSpec(s, lambda i: (0,) * len(s))
    return pl.pallas_call(
        _edge1_kernel,
        grid=(EP // _BE,),
        in_specs=[rows(4), rows(1), rows(1),
                  pl.BlockSpec((1, 1, _BE), lambda i: (i, 0, 0)), rows(128),
                  full((4, 32)), full((1, 32)), full((32, 64)),
                  full((1, 64)), full((8, 128)), full((64, 128)),
                  full((16, 128))],
        out_specs=(rows(128), full((40, 136))),
        out_shape=(jax.ShapeDtypeStruct((EP, 128), jnp.float32),
                   jax.ShapeDtypeStruct((40, 136), jnp.float32)),
    )(ea, src2, dst2, bsrc3, g, p['enc_edge_w1'], p['enc_edge_b1'][None],
      p['enc_edge_w2'], p['enc_edge_b2'][None], tsep, wenc, wrbf)


def _edgek_kernel(e_ref, bsrc_ref, g_ref, w_ref, out_ref, eg_ref):
    e_new = jnp.maximum(_mm(e_ref[...], w_ref[...]) + g_ref[...], 0.0)
    out_ref[...] = e_new
    _eg_accum(pl.program_id(0), bsrc_ref, e_new, eg_ref)


def _edgek(e, bsrc3, g, wee):
    EP = e.shape[0]
    rows = lambda d: pl.BlockSpec((_BE, d), lambda i: (i, 0))
    return pl.pallas_call(
        _edgek_kernel,
        grid=(EP // _BE,),
        in_specs=[rows(128),
                  pl.BlockSpec((1, 1, _BE), lambda i: (i, 0, 0)), rows(128),
                  pl.BlockSpec((128, 128), lambda i: (0, 0))],
        out_specs=(rows(128), pl.BlockSpec((40, 136), lambda i: (0, 0))),
        out_shape=(jax.ShapeDtypeStruct((EP, 128), jnp.float32),
                   jax.ShapeDtypeStruct((40, 136), jnp.float32)),
    )(e, bsrc3, g, wee)


def _node_core(xh_ref, msga_ref, msgb_ref, deg_ref, b_ref, bt_ref, cntn_ref,
               u_ref, eg_ref, cnte_ref, wnx_ref, wnm_ref, wnu_ref,
               bn_ref, wge_ref, wgx_ref, wgu_ref, bg_ref):
    B = b_ref[...]
    u = u_ref[...]
    msg = (msga_ref[...] + msgb_ref[...]) / deg_ref[...]
    xh_new = jnp.maximum(
        _mm(xh_ref[...], wnx_ref[...]) + _mm(msg, wnm_ref[...])
        + _mm(B, _mm(u, wnu_ref[...])) + bn_ref[...], 0.0)
    xg = _mm(bt_ref[...], xh_new) / cntn_ref[...]
    eg = eg_ref[...] / cnte_ref[...]
    u_new = jnp.maximum(
        _mm(eg, wge_ref[...]) + _mm(xg, wgx_ref[...])
        + _mm(u, wgu_ref[...]) + bg_ref[...], 0.0)
    return B, xh_new, u_new


def _node_upd_kernel(xh_ref, msga_ref, msgb_ref, deg_ref, b_ref, bt_ref,
                     cntn_ref, u_ref, eg_ref, cnte_ref,
                     wnx_ref, wnm_ref, wnu_ref,
                     bn_ref, wge_ref, wgx_ref, wgu_ref, bg_ref,
                     wpx_ref, wpu_ref, bp_ref,
                     xh_out_ref, u_out_ref, p_out_ref):
    B, xh_new, u_new = _node_core(
        xh_ref, msga_ref, msgb_ref, deg_ref, b_ref, bt_ref, cntn_ref, u_ref,
        eg_ref, cnte_ref, wnx_ref, wnm_ref, wnu_ref, bn_ref,
        wge_ref, wgx_ref, wgu_ref, bg_ref)
    xh_out_ref[...] = xh_new
    u_out_ref[...] = u_new
    p_out_ref[...] = (_mm(xh_new, wpx_ref[...])
                      + _mm(B, _mm(u_new, wpu_ref[...])) + bp_ref[...])


def _node_final_kernel(xh_ref, msga_ref, msgb_ref, deg_ref, b_ref, bt_ref,
                       cntn_ref, u_ref, eg_ref, cnte_ref,
                       wnx_ref, wnm_ref, wnu_ref,
                       bn_ref, wge_ref, wgx_ref, wgu_ref, bg_ref,
                       rw_ref, rb_ref, gw_ref, gb_ref,
                       x_out_ref, u_out_ref):
    _, xh_new, u_new = _node_core(
        xh_ref, msga_ref, msgb_ref, deg_ref, b_ref, bt_ref, cntn_ref, u_ref,
        eg_ref, cnte_ref, wnx_ref, wnm_ref, wnu_ref, bn_ref,
        wge_ref, wgx_ref, wgu_ref, bg_ref)
    x_out_ref[...] = _sig(_mm(xh_new, rw_ref[...]) + rb_ref[...])
    u_out_ref[...] = _sig(_mm(u_new, gw_ref[...]) + gb_ref[...])


def _split_layer_weights(p, k, dx):
    mp = p['mp'][k]
    no = mp['node_w'].shape[1]
    wnx = mp['node_w'][:dx]
    wnm = mp['node_w'][dx:dx + 128]
    wge = mp['global_w'][:128]
    wgx = mp['global_w'][128:128 + no]
    if k == 0:
        wnu = jnp.zeros((32, no), jnp.float32)
        wgu = jnp.zeros((32, 32), jnp.float32)
    else:
        wnu = mp['node_w'][dx + 128:]
        wgu = mp['global_w'][128 + no:]
    return wnx, wnm, wnu, mp['node_b'][None], wge, wgx, wgu, mp['global_b'][None]


def _node_update(xh, msgp, deg2, B, BT, cntn2, u, eg, cnte2, p, k):
    N = xh.shape[0]
    dx = xh.shape[1]
    ws = _split_layer_weights(p, k, dx)
    no = ws[0].shape[1]
    args = (xh, msgp[0], msgp[1], deg2, B, BT, cntn2, u, eg,
            cnte2) + ws
    if k < 3:
        mpn = p['mp'][k + 1]
        wpx = mpn['edge_w'][:no]
        wpu = mpn['edge_w'][no + 128:]
        bp = mpn['edge_b'][None]
        return pl.pallas_call(
            _node_upd_kernel,
            out_shape=(jax.ShapeDtypeStruct((N, no), jnp.float32),
                       jax.ShapeDtypeStruct((32, 32), jnp.float32),
                       jax.ShapeDtypeStruct((N, 128), jnp.float32)),
        )(*args, wpx, wpu, bp)
    return pl.pallas_call(
        _node_final_kernel,
        out_shape=(jax.ShapeDtypeStruct((N, 2), jnp.float32),
                   jax.ShapeDtypeStruct((32, 5), jnp.float32)),
    )(*args, p['ro_node_w'], p['ro_node_b'][None],
      p['ro_global_w'], p['ro_global_b'][None])


_NW = 32
_CH = 128


def _sc_gather(P, idx2d):
    EP = idx2d.shape[0] * _CH
    per_w = EP // _NW
    k = per_w // _CH

    @functools.partial(
        pl.kernel,
        mesh=plsc.VectorSubcoreMesh(core_axis_name="c", subcore_axis_name="s"),
        out_type=jax.ShapeDtypeStruct((EP, 128), jnp.float32),
        scratch_types=[
            pltpu.VMEM((k, _CH), jnp.int32),
            pltpu.VMEM((_CH, 128), jnp.float32),
            pltpu.VMEM((_CH, 128), jnp.float32),
            pltpu.VMEM((_CH, 128), jnp.float32),
            pltpu.VMEM((_CH, 128), jnp.float32),
            pltpu.SemaphoreType.DMA,
            pltpu.SemaphoreType.DMA,
            pltpu.SemaphoreType.DMA,
            pltpu.SemaphoreType.DMA,
        ],
    )
    def gk(p_hbm, idx_hbm, out_hbm, idx_v,
           rows_v0, rows_v1, rows_v2, rows_v3, sem0, sem1, sem2, sem3):
        wid = lax.axis_index("s") * 2 + lax.axis_index("c")
        pltpu.sync_copy(idx_hbm.at[pl.ds(wid * k, k)], idx_v)
        rows = (rows_v0, rows_v1, rows_v2, rows_v3)
        sems = (sem0, sem1, sem2, sem3)
        base = wid * per_w
        pltpu.async_copy(p_hbm.at[idx_v.at[0]], rows[0], sems[0])
        pltpu.async_copy(p_hbm.at[idx_v.at[1]], rows[1], sems[1])

        def wait_any(b):
            pltpu.make_async_copy(p_hbm.at[idx_v.at[0]], rows[b],
                                  sems[b]).wait()

        def body(t, _):
            for b in range(4):
                j = 4 * t + b
                wait_any(b)
                pltpu.async_copy(rows[b],
                                 out_hbm.at[pl.ds(base + j * _CH, _CH)],
                                 sems[b])
                b2 = (b + 2) % 4
                if b >= 2:
                    wait_any(b2)
                else:
                    @pl.when(t > 0)
                    def _():
                        wait_any(b2)

                jn = jnp.minimum(j + 2, k - 1)
                pltpu.async_copy(p_hbm.at[idx_v.at[jn]], rows[b2], sems[b2])
            return 0

        lax.fori_loop(0, k // 4, body, 0)
        for b in range(2):
            wait_any(b)
        for b in range(2):
            wait_any((k - 2 + b) % 4)

    return gk(P, idx2d)


def _sc_scatter(e, dst2d, init, n_rows):
    EP = e.shape[0]
    per_w = EP // _NW
    k = per_w // _CH
    NA = ((n_rows + 8 + 631) // 632) * 632
    wr = 632
    nw_full = n_rows // wr
    tail = n_rows - nw_full * wr

    @functools.partial(
        pl.kernel,
        mesh=plsc.VectorSubcoreMesh(core_axis_name="c", subcore_axis_name="s"),
        out_type=jax.ShapeDtypeStruct((2, n_rows, 128), jnp.float32),
        scratch_types=[
            pltpu.VMEM((k, _CH), jnp.int32),
            pltpu.VMEM((_CH, 128), jnp.float32),
            pltpu.VMEM((_CH, 128), jnp.float32),
            pltpu.VMEM_SHARED((NA, 128), jnp.float32),
            pltpu.SemaphoreType.DMA,
            pltpu.SemaphoreType.DMA,
        ],
    )
    def sk(e_hbm, d_hbm, i_hbm, msg_hbm, di_v,
           rows_v0, rows_v1, acc_sh, sem0, sem1):
        c = lax.axis_index("c")
        s = lax.axis_index("s")
        wid = s * 2 + c
        rows = (rows_v0, rows_v1)
        sems = (sem0, sem1)
        pltpu.sync_copy(d_hbm.at[pl.ds(wid * k, k)], di_v)
        @pl.when(s < nw_full)
        def _():
            pltpu.sync_copy(i_hbm.at[c].at[pl.ds(s * wr, wr)],
                            acc_sh.at[pl.ds(s * wr, wr)])

        @pl.when(s == nw_full)
        def _():
            pltpu.sync_copy(i_hbm.at[c].at[pl.ds(nw_full * wr, tail)],
                            acc_sh.at[pl.ds(nw_full * wr, tail)])

        plsc.subcore_barrier()
        base = wid * per_w
        pltpu.async_copy(e_hbm.at[pl.ds(base, _CH)], rows[0], sems[0])
        pltpu.async_copy(e_hbm.at[pl.ds(base + _CH, _CH)], rows[1], sems[1])

        def body(t, _):
            for b in range(2):
                j = 2 * t + b
                pltpu.make_async_copy(e_hbm.at[pl.ds(0, _CH)], rows[b],
                                      sems[b]).wait()
                pltpu.sync_copy(rows[b], acc_sh.at[di_v.at[j]], add=True)
                jn = jnp.minimum(j + 2, k - 1)
                pltpu.async_copy(e_hbm.at[pl.ds(base + jn * _CH, _CH)],
                                 rows[b], sems[b])
            return 0

        lax.fori_loop(0, k // 2, body, 0)
        for b in range(2):
            pltpu.make_async_copy(e_hbm.at[pl.ds(0, _CH)], rows[b],
                                  sems[b]).wait()
        plsc.subcore_barrier()
        @pl.when(s < nw_full)
        def _():
            pltpu.sync_copy(acc_sh.at[pl.ds(s * wr, wr)],
                            msg_hbm.at[c].at[pl.ds(s * wr, wr)])

        @pl.when(s == nw_full)
        def _():
            pltpu.sync_copy(acc_sh.at[pl.ds(nw_full * wr, tail)],
                            msg_hbm.at[c].at[pl.ds(nw_full * wr, tail)])

    return sk(e, dst2d, init)


def kernel(aa, msa_feats, x, edge_index, edge_attr, secondary_structure,
           batch, params):
    p = params
    N = aa.shape[0]
    E = edge_index.shape[1]
    EP = ((E + 32767) // 32768) * 32768
    pad = EP - E

    src, dst = edge_index[0], edge_index[1]
    bsrc = batch[src]
    srcp = jnp.concatenate([src, jnp.zeros((pad,), src.dtype)])
    dstp = jnp.concatenate([dst, jnp.full((pad,), N, dst.dtype)])
    bsrcp = jnp.concatenate([bsrc, jnp.full((pad,), _NG, bsrc.dtype)])
    eap = jnp.concatenate([edge_attr, jnp.zeros((pad, 4), jnp.float32)])
    src2 = srcp[:, None]
    dst2 = dstp[:, None]
    src2d = srcp.reshape(EP // _CH, _CH).astype(jnp.int32)
    dst2d = dstp.reshape(EP // _CH, _CH).astype(jnp.int32)
    bsrc3 = bsrcp.reshape(EP // _BE, 1, _BE).astype(jnp.int32)
    zeros2 = jnp.zeros((2, N, 128), jnp.float32)
    EH = EP
    h2d = EH // _CH
    h3d = EH // _BE

    deg2 = jnp.clip(jax.ops.segment_sum(jnp.ones((E,), jnp.float32), dst,
                                        num_segments=N), 1.0)[:, None]
    B = (batch[:, None] == jnp.arange(_NG)[None, :]).astype(jnp.float32)
    BT = B.T
    cntn2 = jnp.clip(jnp.sum(B, axis=0), 1.0)[:, None]

    nodes = jnp.concatenate([x, msa_feats], axis=1)
    xh, P = _node_encoder(nodes, aa[:, None].astype(jnp.int32),
                          secondary_structure[:, None].astype(jnp.int32), p)
    u = jnp.zeros((32, 32), jnp.float32)

    eh = [None, None]
    for k in range(4):
        eg40s = []
        msgp = zeros2
        for h in range(1):
            g = _sc_gather(P, src2d[h * h2d:(h + 1) * h2d])
            if k == 0:
                ehh, eg40 = _edge1(eap[h * EH:(h + 1) * EH],
                                   src2[h * EH:(h + 1) * EH],
                                   dst2[h * EH:(h + 1) * EH],
                                   bsrc3[h * h3d:(h + 1) * h3d], g, p)
            else:
                dx = xh.shape[1]
                wee = p['mp'][k]['edge_w'][dx:dx + 128]
                ehh, eg40 = _edgek(eh[h], bsrc3[h * h3d:(h + 1) * h3d],
                                   g, wee)
            eh[h] = ehh
            eg40s.append(eg40)
            msgp = _sc_scatter(ehh, dst2d[h * h2d:(h + 1) * h2d], msgp, N)
        eg40 = sum(eg40s)
        eg = eg40[:_NG, :128]
        cnte2 = jnp.clip(eg40[:_NG, 128:129], 1.0)
        out = _node_update(xh, msgp, deg2, B, BT, cntn2, u, eg,
                           cnte2, p, k)
        if k < 3:
            xh, u, P = out
        else:
            return out[0], out[1]


# speedup vs baseline: 1.0539x; 1.0539x over previous
import functools
import jax
import jax.numpy as jnp
from jax import lax
from jax.experimental import pallas as pl
from jax.experimental.pallas import tpu as pltpu
from jax.experimental.pallas import tpu_sc as plsc

_NG = 32
_BE = 4096


def _mm(a, b):
    return jax.lax.dot_general(a, b, (((1,), (0,)), ((), ())),
                               preferred_element_type=jnp.float32)


def _sig(v):
    return 1.0 / (1.0 + jnp.exp(-v))


def _node_enc_kernel(nodes_ref, aa_ref, ss_ref, w1_ref, b1_ref, w2_ref,
                     b2_ref, ea_ref, es_ref, wex_ref, be_ref,
                     xh_ref, p_ref):
    h = jnp.maximum(_mm(nodes_ref[...], w1_ref[...]) + b1_ref[...], 0.0)
    h = jnp.maximum(_mm(h, w2_ref[...]) + b2_ref[...], 0.0)
    oh_aa = (aa_ref[...] == jax.lax.broadcasted_iota(jnp.int32, (1, 24), 1)
             ).astype(jnp.float32)
    oh_ss = (ss_ref[...] == jax.lax.broadcasted_iota(jnp.int32, (1, 16), 1)
             ).astype(jnp.float32)
    xh = jnp.concatenate(
        [_mm(oh_aa, ea_ref[...]), h, _mm(oh_ss, es_ref[...])], axis=1)
    xh_ref[...] = xh
    p_ref[...] = _mm(xh, wex_ref[...]) + be_ref[...]


def _node_encoder(nodes, aa2, ss2, p):
    N = nodes.shape[0]
    emb_aa = jnp.zeros((24, 64), jnp.float32).at[:20].set(p['emb_aa'])
    emb_ss = jnp.zeros((16, 16), jnp.float32).at[:9].set(p['emb_ss'])
    wex = p['mp'][0]['edge_w'][:208]
    be = p['mp'][0]['edge_b'][None, :]
    return pl.pallas_call(
        _node_enc_kernel,
        out_shape=(jax.ShapeDtypeStruct((N, 208), jnp.float32),
                   jax.ShapeDtypeStruct((N, 128), jnp.float32)),
    )(nodes, aa2, ss2, p['enc_node_w1'], p['enc_node_b1'][None],
      p['enc_node_w2'], p['enc_node_b2'][None], emb_aa, emb_ss, wex, be)


def _eg_accum(i, bsrc_ref, e_new, eg_ref):
    ohbt = (bsrc_ref[0] == jax.lax.broadcasted_iota(jnp.int32, (40, 1), 0)
            ).astype(jnp.float32)
    ones = jnp.ones((e_new.shape[0], 8), jnp.float32)
    egp = _mm(ohbt, jnp.concatenate([e_new, ones], axis=1))

    @pl.when(i == 0)
    def _():
        eg_ref[...] = jnp.zeros_like(eg_ref)

    eg_ref[...] += egp


def _edge1_kernel(ea_ref, src_ref, dst_ref, bsrc_ref, g_ref, ew1_ref,
                  eb1_ref, ew2_ref, eb2_ref, tsep_ref, wenc_ref, wrbf_ref,
                  out_ref, eg_ref):
    ea = ea_ref[...]
    a = jnp.maximum(_mm(ea, ew1_ref[...]) + eb1_ref[...], 0.0)
    b = jnp.maximum(_mm(a, ew2_ref[...]) + eb2_ref[...], 0.0)
    acc = _mm(b, wenc_ref[...])
    d = jnp.abs(src_ref[...] - dst_ref[...])
    code = ((d > 1).astype(jnp.int32) + (d > 2) + (d > 3) + (d > 4)
            + (d > 5) + (d > 10) + (d > 15))
    oh = (code == jax.lax.broadcasted_iota(jnp.int32, (1, 8), 1)
          ).astype(jnp.float32)
    acc = acc + _mm(oh, tsep_ref[...])
    cen = jax.lax.broadcasted_iota(jnp.int32, (1, 16), 1).astype(
        jnp.float32) * (20.0 / 15.0)
    rbf = jnp.exp(-(ea[:, 0:1] - cen) ** 2)
    acc = acc + _mm(rbf, wrbf_ref[...])
    e_new = jnp.maximum(acc + g_ref[...], 0.0)
    out_ref[...] = e_new
    _eg_accum(pl.program_id(0), bsrc_ref, e_new, eg_ref)


def _edge1(ea, src2, dst2, bsrc3, g, p):
    EP = ea.shape[0]
    mp = p['mp'][0]
    wenc = mp['edge_w'][208:272]
    tsep = p['emb_sep'] @ mp['edge_w'][272:304]
    wrbf = mp['edge_w'][304:320]
    rows = lambda d: pl.BlockSpec((_BE, d), lambda i: (i, 0))
    full = lambda s: pl.BlockSpec(s, lambda i: (0,) * len(s))
    return pl.pallas_call(
        _edge1_kernel,
        grid=(EP // _BE,),
        in_specs=[rows(4), rows(1), rows(1),
                  pl.BlockSpec((1, 1, _BE), lambda i: (i, 0, 0)), rows(128),
                  full((4, 32)), full((1, 32)), full((32, 64)),
                  full((1, 64)), full((8, 128)), full((64, 128)),
                  full((16, 128))],
        out_specs=(rows(128), full((40, 136))),
        out_shape=(jax.ShapeDtypeStruct((EP, 128), jnp.float32),
                   jax.ShapeDtypeStruct((40, 136), jnp.float32)),
    )(ea, src2, dst2, bsrc3, g, p['enc_edge_w1'], p['enc_edge_b1'][None],
      p['enc_edge_w2'], p['enc_edge_b2'][None], tsep, wenc, wrbf)


def _edgek_kernel(e_ref, bsrc_ref, g_ref, w_ref, out_ref, eg_ref):
    e_new = jnp.maximum(_mm(e_ref[...], w_ref[...]) + g_ref[...], 0.0)
    out_ref[...] = e_new
    _eg_accum(pl.program_id(0), bsrc_ref, e_new, eg_ref)


def _edgek(e, bsrc3, g, wee):
    EP = e.shape[0]
    rows = lambda d: pl.BlockSpec((_BE, d), lambda i: (i, 0))
    return pl.pallas_call(
        _edgek_kernel,
        grid=(EP // _BE,),
        in_specs=[rows(128),
                  pl.BlockSpec((1, 1, _BE), lambda i: (i, 0, 0)), rows(128),
                  pl.BlockSpec((128, 128), lambda i: (0, 0))],
        out_specs=(rows(128), pl.BlockSpec((40, 136), lambda i: (0, 0))),
        out_shape=(jax.ShapeDtypeStruct((EP, 128), jnp.float32),
                   jax.ShapeDtypeStruct((40, 136), jnp.float32)),
    )(e, bsrc3, g, wee)


def _node_core(xh_ref, msga_ref, msgb_ref, deg_ref, b_ref, bt_ref, cntn_ref,
               u_ref, eg_ref, cnte_ref, wnx_ref, wnm_ref, wnu_ref,
               bn_ref, wge_ref, wgx_ref, wgu_ref, bg_ref):
    B = b_ref[...]
    u = u_ref[...]
    msg = (msga_ref[...] + msgb_ref[...]) / deg_ref[...]
    xh_new = jnp.maximum(
        _mm(xh_ref[...], wnx_ref[...]) + _mm(msg, wnm_ref[...])
        + _mm(B, _mm(u, wnu_ref[...])) + bn_ref[...], 0.0)
    xg = _mm(bt_ref[...], xh_new) / cntn_ref[...]
    eg = eg_ref[...] / cnte_ref[...]
    u_new = jnp.maximum(
        _mm(eg, wge_ref[...]) + _mm(xg, wgx_ref[...])
        + _mm(u, wgu_ref[...]) + bg_ref[...], 0.0)
    return B, xh_new, u_new


def _node_upd_kernel(xh_ref, msga_ref, msgb_ref, deg_ref, b_ref, bt_ref,
                     cntn_ref, u_ref, eg_ref, cnte_ref,
                     wnx_ref, wnm_ref, wnu_ref,
                     bn_ref, wge_ref, wgx_ref, wgu_ref, bg_ref,
                     wpx_ref, wpu_ref, bp_ref,
                     xh_out_ref, u_out_ref, p_out_ref):
    B, xh_new, u_new = _node_core(
        xh_ref, msga_ref, msgb_ref, deg_ref, b_ref, bt_ref, cntn_ref, u_ref,
        eg_ref, cnte_ref, wnx_ref, wnm_ref, wnu_ref, bn_ref,
        wge_ref, wgx_ref, wgu_ref, bg_ref)
    xh_out_ref[...] = xh_new
    u_out_ref[...] = u_new
    p_out_ref[...] = (_mm(xh_new, wpx_ref[...])
                      + _mm(B, _mm(u_new, wpu_ref[...])) + bp_ref[...])


def _node_final_kernel(xh_ref, msga_ref, msgb_ref, deg_ref, b_ref, bt_ref,
                       cntn_ref, u_ref, eg_ref, cnte_ref,
                       wnx_ref, wnm_ref, wnu_ref,
                       bn_ref, wge_ref, wgx_ref, wgu_ref, bg_ref,
                       rw_ref, rb_ref, gw_ref, gb_ref,
                       x_out_ref, u_out_ref):
    _, xh_new, u_new = _node_core(
        xh_ref, msga_ref, msgb_ref, deg_ref, b_ref, bt_ref, cntn_ref, u_ref,
        eg_ref, cnte_ref, wnx_ref, wnm_ref, wnu_ref, bn_ref,
        wge_ref, wgx_ref, wgu_ref, bg_ref)
    x_out_ref[...] = _sig(_mm(xh_new, rw_ref[...]) + rb_ref[...])
    u_out_ref[...] = _sig(_mm(u_new, gw_ref[...]) + gb_ref[...])


def _split_layer_weights(p, k, dx):
    mp = p['mp'][k]
    no = mp['node_w'].shape[1]
    wnx = mp['node_w'][:dx]
    wnm = mp['node_w'][dx:dx + 128]
    wge = mp['global_w'][:128]
    wgx = mp['global_w'][128:128 + no]
    if k == 0:
        wnu = jnp.zeros((32, no), jnp.float32)
        wgu = jnp.zeros((32, 32), jnp.float32)
    else:
        wnu = mp['node_w'][dx + 128:]
        wgu = mp['global_w'][128 + no:]
    return wnx, wnm, wnu, mp['node_b'][None], wge, wgx, wgu, mp['global_b'][None]


def _node_update(xh, msgp, deg2, B, BT, cntn2, u, eg, cnte2, p, k):
    N = xh.shape[0]
    dx = xh.shape[1]
    ws = _split_layer_weights(p, k, dx)
    no = ws[0].shape[1]
    args = (xh, msgp[0], msgp[1], deg2, B, BT, cntn2, u, eg,
            cnte2) + ws
    if k < 3:
        mpn = p['mp'][k + 1]
        wpx = mpn['edge_w'][:no]
        wpu = mpn['edge_w'][no + 128:]
        bp = mpn['edge_b'][None]
        return pl.pallas_call(
            _node_upd_kernel,
            out_shape=(jax.ShapeDtypeStruct((N, no), jnp.float32),
                       jax.ShapeDtypeStruct((32, 32), jnp.float32),
                       jax.ShapeDtypeStruct((N, 128), jnp.float32)),
        )(*args, wpx, wpu, bp)
    return pl.pallas_call(
        _node_final_kernel,
        out_shape=(jax.ShapeDtypeStruct((N, 2), jnp.float32),
                   jax.ShapeDtypeStruct((32, 5), jnp.float32)),
    )(*args, p['ro_node_w'], p['ro_node_b'][None],
      p['ro_global_w'], p['ro_global_b'][None])


_NW = 32
_CH = 128


def _sc_gather(P, idx2d):
    EP = idx2d.shape[0] * _CH
    per_w = EP // _NW
    k = per_w // _CH

    @functools.partial(
        pl.kernel,
        mesh=plsc.VectorSubcoreMesh(core_axis_name="c", subcore_axis_name="s"),
        out_type=jax.ShapeDtypeStruct((EP, 128), jnp.float32),
        scratch_types=[
            pltpu.VMEM((k, _CH), jnp.int32),
            pltpu.VMEM((_CH, 128), jnp.float32),
            pltpu.VMEM((_CH, 128), jnp.float32),
            pltpu.VMEM((_CH, 128), jnp.float32),
            pltpu.VMEM((_CH, 128), jnp.float32),
            pltpu.SemaphoreType.DMA,
            pltpu.SemaphoreType.DMA,
            pltpu.SemaphoreType.DMA,
            pltpu.SemaphoreType.DMA,
        ],
    )
    def gk(p_hbm, idx_hbm, out_hbm, idx_v,
           rows_v0, rows_v1, rows_v2, rows_v3, sem0, sem1, sem2, sem3):
        wid = lax.axis_index("s") * 2 + lax.axis_index("c")
        pltpu.sync_copy(idx_hbm.at[pl.ds(wid * k, k)], idx_v)
        rows = (rows_v0, rows_v1, rows_v2, rows_v3)
        sems = (sem0, sem1, sem2, sem3)
        base = wid * per_w
        pltpu.async_copy(p_hbm.at[idx_v.at[0]], rows[0], sems[0])
        pltpu.async_copy(p_hbm.at[idx_v.at[1]], rows[1], sems[1])

        def wait_any(b):
            pltpu.make_async_copy(p_hbm.at[idx_v.at[0]], rows[b],
                                  sems[b]).wait()

        def body(t, _):
            for b in range(4):
                j = 4 * t + b
                wait_any(b)
                pltpu.async_copy(rows[b],
                                 out_hbm.at[pl.ds(base + j * _CH, _CH)],
                                 sems[b])
                b2 = (b + 2) % 4
                if b >= 2:
                    wait_any(b2)
                else:
                    @pl.when(t > 0)
                    def _():
                        wait_any(b2)

                jn = jnp.minimum(j + 2, k - 1)
                pltpu.async_copy(p_hbm.at[idx_v.at[jn]], rows[b2], sems[b2])
            return 0

        lax.fori_loop(0, k // 4, body, 0)
        for b in range(2):
            wait_any(b)
        for b in range(2):
            wait_any((k - 2 + b) % 4)

    return gk(P, idx2d)


def _sc_scatter(e, dst2d, init, n_rows):
    EP = e.shape[0]
    per_w = EP // _NW
    k = per_w // _CH
    NA = ((n_rows + 8 + 631) // 632) * 632
    wr = 632
    nw_full = n_rows // wr
    tail = n_rows - nw_full * wr

    @functools.partial(
        pl.kernel,
        mesh=plsc.VectorSubcoreMesh(core_axis_name="c", subcore_axis_name="s"),
        out_type=jax.ShapeDtypeStruct((2, n_rows, 128), jnp.float32),
        scratch_types=[
            pltpu.VMEM((k, _CH), jnp.int32),
            pltpu.VMEM((_CH, 128), jnp.float32),
            pltpu.VMEM((_CH, 128), jnp.float32),
            pltpu.VMEM_SHARED((NA, 128), jnp.float32),
            pltpu.SemaphoreType.DMA,
            pltpu.SemaphoreType.DMA,
        ],
    )
    def sk(e_hbm, d_hbm, i_hbm, msg_hbm, di_v,
           rows_v0, rows_v1, acc_sh, sem0, sem1):
        c = lax.axis_index("c")
        s = lax.axis_index("s")
        wid = s * 2 + c
        rows = (rows_v0, rows_v1)
        sems = (sem0, sem1)
        pltpu.sync_copy(d_hbm.at[pl.ds(wid * k, k)], di_v)
        @pl.when(s < nw_full)
        def _():
            pltpu.sync_copy(i_hbm.at[c].at[pl.ds(s * wr, wr)],
                            acc_sh.at[pl.ds(s * wr, wr)])

        @pl.when(s == nw_full)
        def _():
            pltpu.sync_copy(i_hbm.at[c].at[pl.ds(nw_full * wr, tail)],
                            acc_sh.at[pl.ds(nw_full * wr, tail)])

        plsc.subcore_barrier()
        base = wid * per_w
        pltpu.async_copy(e_hbm.at[pl.ds(base, _CH)], rows[0], sems[0])
        pltpu.async_copy(e_hbm.at[pl.ds(base + _CH, _CH)], rows[1], sems[1])

        def body(t, _):
            for b in range(2):
                j = 2 * t + b
                pltpu.make_async_copy(e_hbm.at[pl.ds(0, _CH)], rows[b],
                                      sems[b]).wait()
                pltpu.sync_copy(rows[b], acc_sh.at[di_v.at[j]], add=True)
                jn = jnp.minimum(j + 2, k - 1)
                pltpu.async_copy(e_hbm.at[pl.ds(base + jn * _CH, _CH)],
                                 rows[b], sems[b])
            return 0

        lax.fori_loop(0, k // 2, body, 0)
        for b in range(2):
            pltpu.make_async_copy(e_hbm.at[pl.ds(0, _CH)], rows[b],
                                  sems[b]).wait()
        plsc.subcore_barrier()
        @pl.when(s < nw_full)
        def _():
            pltpu.sync_copy(acc_sh.at[pl.ds(s * wr, wr)],
                            msg_hbm.at[c].at[pl.ds(s * wr, wr)])

        @pl.when(s == nw_full)
        def _():
            pltpu.sync_copy(acc_sh.at[pl.ds(nw_full * wr, tail)],
                            msg_hbm.at[c].at[pl.ds(nw_full * wr, tail)])

    return sk(e, dst2d, init)


def kernel(aa, msa_feats, x, edge_index, edge_attr, secondary_structure,
           batch, params):
    p = params
    N = aa.shape[0]
    E = edge_index.shape[1]
    EP = ((E + 32767) // 32768) * 32768
    pad = EP - E

    src, dst = edge_index[0], edge_index[1]
    bsrc = batch[src]
    srcp = jnp.concatenate([src, jnp.zeros((pad,), src.dtype)])
    dstp = jnp.concatenate([dst, jnp.full((pad,), N, dst.dtype)])
    bsrcp = jnp.concatenate([bsrc, jnp.full((pad,), _NG, bsrc.dtype)])
    eap = jnp.concatenate([edge_attr, jnp.zeros((pad, 4), jnp.float32)])
    src2 = srcp[:, None]
    dst2 = dstp[:, None]
    src2d = srcp.reshape(EP // _CH, _CH).astype(jnp.int32)
    dst2d = dstp.reshape(EP // _CH, _CH).astype(jnp.int32)
    bsrc3 = bsrcp.reshape(EP // _BE, 1, _BE).astype(jnp.int32)
    zeros2 = jnp.zeros((2, N, 128), jnp.float32)
    EH = 65536
    h2d = EH // _CH
    h3d = EH // _BE

    deg2 = jnp.clip(jax.ops.segment_sum(jnp.ones((E,), jnp.float32), dst,
                                        num_segments=N), 1.0)[:, None]
    B = (batch[:, None] == jnp.arange(_NG)[None, :]).astype(jnp.float32)
    BT = B.T
    cntn2 = jnp.clip(jnp.sum(B, axis=0), 1.0)[:, None]

    nodes = jnp.concatenate([x, msa_feats], axis=1)
    xh, P = _node_encoder(nodes, aa[:, None].astype(jnp.int32),
                          secondary_structure[:, None].astype(jnp.int32), p)
    u = jnp.zeros((32, 32), jnp.float32)

    eh = [None] * (EP // EH)
    for k in range(4):
        eg40s = []
        msgp = zeros2
        for h in range(EP // EH):
            g = _sc_gather(P, src2d[h * h2d:(h + 1) * h2d])
            if k == 0:
                ehh, eg40 = _edge1(eap[h * EH:(h + 1) * EH],
                                   src2[h * EH:(h + 1) * EH],
                                   dst2[h * EH:(h + 1) * EH],
                                   bsrc3[h * h3d:(h + 1) * h3d], g, p)
            else:
                dx = xh.shape[1]
                wee = p['mp'][k]['edge_w'][dx:dx + 128]
                ehh, eg40 = _edgek(eh[h], bsrc3[h * h3d:(h + 1) * h3d],
                                   g, wee)
            eh[h] = ehh
            eg40s.append(eg40)
            msgp = _sc_scatter(ehh, dst2d[h * h2d:(h + 1) * h2d], msgp, N)
        eg40 = sum(eg40s)
        eg = eg40[:_NG, :128]
        cnte2 = jnp.clip(eg40[:_NG, 128:129], 1.0)
        out = _node_update(xh, msgp, deg2, B, BT, cntn2, u, eg,
                           cnte2, p, k)
        if k < 3:
            xh, u, P = out
        else:
            return out[0], out[1]


# speedup vs baseline: 1.0881x; 1.0325x over previous
import functools
import jax
import jax.numpy as jnp
from jax import lax
from jax.experimental import pallas as pl
from jax.experimental.pallas import tpu as pltpu
from jax.experimental.pallas import tpu_sc as plsc

_NG = 32
_BE = 4096


def _mm(a, b):
    return jax.lax.dot_general(a, b, (((1,), (0,)), ((), ())),
                               preferred_element_type=jnp.float32)


def _sig(v):
    return 1.0 / (1.0 + jnp.exp(-v))


def _node_enc_kernel(nodes_ref, aa_ref, ss_ref, w1_ref, b1_ref, w2_ref,
                     b2_ref, ea_ref, es_ref, wex_ref, be_ref,
                     xh_ref, p_ref):
    h = jnp.maximum(_mm(nodes_ref[...], w1_ref[...]) + b1_ref[...], 0.0)
    h = jnp.maximum(_mm(h, w2_ref[...]) + b2_ref[...], 0.0)
    oh_aa = (aa_ref[...] == jax.lax.broadcasted_iota(jnp.int32, (1, 24), 1)
             ).astype(jnp.float32)
    oh_ss = (ss_ref[...] == jax.lax.broadcasted_iota(jnp.int32, (1, 16), 1)
             ).astype(jnp.float32)
    xh = jnp.concatenate(
        [_mm(oh_aa, ea_ref[...]), h, _mm(oh_ss, es_ref[...])], axis=1)
    xh_ref[...] = xh
    p_ref[...] = _mm(xh, wex_ref[...]) + be_ref[...]


def _node_encoder(nodes, aa2, ss2, p):
    N = nodes.shape[0]
    emb_aa = jnp.zeros((24, 64), jnp.float32).at[:20].set(p['emb_aa'])
    emb_ss = jnp.zeros((16, 16), jnp.float32).at[:9].set(p['emb_ss'])
    wex = p['mp'][0]['edge_w'][:208]
    be = p['mp'][0]['edge_b'][None, :]
    return pl.pallas_call(
        _node_enc_kernel,
        out_shape=(jax.ShapeDtypeStruct((N, 208), jnp.float32),
                   jax.ShapeDtypeStruct((N, 128), jnp.float32)),
    )(nodes, aa2, ss2, p['enc_node_w1'], p['enc_node_b1'][None],
      p['enc_node_w2'], p['enc_node_b2'][None], emb_aa, emb_ss, wex, be)


def _eg_accum(i, bsrc_ref, e_new, eg_ref):
    ohbt = (bsrc_ref[0] == jax.lax.broadcasted_iota(jnp.int32, (40, 1), 0)
            ).astype(jnp.float32)
    ones = jnp.ones((e_new.shape[0], 8), jnp.float32)
    egp = _mm(ohbt, jnp.concatenate([e_new, ones], axis=1))

    @pl.when(i == 0)
    def _():
        eg_ref[...] = jnp.zeros_like(eg_ref)

    eg_ref[...] += egp


def _edge1_kernel(ea_ref, src_ref, dst_ref, bsrc_ref, g_ref, ew1_ref,
                  eb1_ref, ew2_ref, eb2_ref, tsep_ref, wenc_ref, wrbf_ref,
                  out_ref, eg_ref):
    ea = ea_ref[...]
    a = jnp.maximum(_mm(ea, ew1_ref[...]) + eb1_ref[...], 0.0)
    b = jnp.maximum(_mm(a, ew2_ref[...]) + eb2_ref[...], 0.0)
    acc = _mm(b, wenc_ref[...])
    d = jnp.abs(src_ref[...] - dst_ref[...])
    code = ((d > 1).astype(jnp.int32) + (d > 2) + (d > 3) + (d > 4)
            + (d > 5) + (d > 10) + (d > 15))
    oh = (code == jax.lax.broadcasted_iota(jnp.int32, (1, 8), 1)
          ).astype(jnp.float32)
    acc = acc + _mm(oh, tsep_ref[...])
    cen = jax.lax.broadcasted_iota(jnp.int32, (1, 16), 1).astype(
        jnp.float32) * (20.0 / 15.0)
    rbf = jnp.exp(-(ea[:, 0:1] - cen) ** 2)
    acc = acc + _mm(rbf, wrbf_ref[...])
    e_new = jnp.maximum(acc + g_ref[...], 0.0)
    out_ref[...] = e_new
    _eg_accum(pl.program_id(0), bsrc_ref, e_new, eg_ref)


def _edge1(ea, src2, dst2, bsrc3, g, p):
    EP = ea.shape[0]
    mp = p['mp'][0]
    wenc = mp['edge_w'][208:272]
    tsep = p['emb_sep'] @ mp['edge_w'][272:304]
    wrbf = mp['edge_w'][304:320]
    rows = lambda d: pl.BlockSpec((_BE, d), lambda i: (i, 0))
    full = lambda s: pl.BlockSpec(s, lambda i: (0,) * len(s))
    return pl.pallas_call(
        _edge1_kernel,
        grid=(EP // _BE,),
        in_specs=[rows(4), rows(1), rows(1),
                  pl.BlockSpec((1, 1, _BE), lambda i: (i, 0, 0)), rows(128),
                  full((4, 32)), full((1, 32)), full((32, 64)),
                  full((1, 64)), full((8, 128)), full((64, 128)),
                  full((16, 128))],
        out_specs=(rows(128), full((40, 136))),
        out_shape=(jax.ShapeDtypeStruct((EP, 128), jnp.float32),
                   jax.ShapeDtypeStruct((40, 136), jnp.float32)),
    )(ea, src2, dst2, bsrc3, g, p['enc_edge_w1'], p['enc_edge_b1'][None],
      p['enc_edge_w2'], p['enc_edge_b2'][None], tsep, wenc, wrbf)


def _edgek_kernel(e_ref, bsrc_ref, g_ref, w_ref, out_ref, eg_ref):
    e_new = jnp.maximum(_mm(e_ref[...], w_ref[...]) + g_ref[...], 0.0)
    out_ref[...] = e_new
    _eg_accum(pl.program_id(0), bsrc_ref, e_new, eg_ref)


def _edgek(e, bsrc3, g, wee):
    EP = e.shape[0]
    rows = lambda d: pl.BlockSpec((_BE, d), lambda i: (i, 0))
    return pl.pallas_call(
        _edgek_kernel,
        grid=(EP // _BE,),
        in_specs=[rows(128),
                  pl.BlockSpec((1, 1, _BE), lambda i: (i, 0, 0)), rows(128),
                  pl.BlockSpec((128, 128), lambda i: (0, 0))],
        out_specs=(rows(128), pl.BlockSpec((40, 136), lambda i: (0, 0))),
        out_shape=(jax.ShapeDtypeStruct((EP, 128), jnp.float32),
                   jax.ShapeDtypeStruct((40, 136), jnp.float32)),
    )(e, bsrc3, g, wee)


def _node_core(xh_ref, msga_ref, msgb_ref, deg_ref, b_ref, bt_ref, cntn_ref,
               u_ref, eg_ref, cnte_ref, wnx_ref, wnm_ref, wnu_ref,
               bn_ref, wge_ref, wgx_ref, wgu_ref, bg_ref):
    B = b_ref[...]
    u = u_ref[...]
    msg = (msga_ref[...] + msgb_ref[...]) / deg_ref[...]
    xh_new = jnp.maximum(
        _mm(xh_ref[...], wnx_ref[...]) + _mm(msg, wnm_ref[...])
        + _mm(B, _mm(u, wnu_ref[...])) + bn_ref[...], 0.0)
    xg = _mm(bt_ref[...], xh_new) / cntn_ref[...]
    eg = eg_ref[...] / cnte_ref[...]
    u_new = jnp.maximum(
        _mm(eg, wge_ref[...]) + _mm(xg, wgx_ref[...])
        + _mm(u, wgu_ref[...]) + bg_ref[...], 0.0)
    return B, xh_new, u_new


def _node_upd_kernel(xh_ref, msga_ref, msgb_ref, deg_ref, b_ref, bt_ref,
                     cntn_ref, u_ref, eg_ref, cnte_ref,
                     wnx_ref, wnm_ref, wnu_ref,
                     bn_ref, wge_ref, wgx_ref, wgu_ref, bg_ref,
                     wpx_ref, wpu_ref, bp_ref,
                     xh_out_ref, u_out_ref, p_out_ref):
    B, xh_new, u_new = _node_core(
        xh_ref, msga_ref, msgb_ref, deg_ref, b_ref, bt_ref, cntn_ref, u_ref,
        eg_ref, cnte_ref, wnx_ref, wnm_ref, wnu_ref, bn_ref,
        wge_ref, wgx_ref, wgu_ref, bg_ref)
    xh_out_ref[...] = xh_new
    u_out_ref[...] = u_new
    p_out_ref[...] = (_mm(xh_new, wpx_ref[...])
                      + _mm(B, _mm(u_new, wpu_ref[...])) + bp_ref[...])


def _node_final_kernel(xh_ref, msga_ref, msgb_ref, deg_ref, b_ref, bt_ref,
                       cntn_ref, u_ref, eg_ref, cnte_ref,
                       wnx_ref, wnm_ref, wnu_ref,
                       bn_ref, wge_ref, wgx_ref, wgu_ref, bg_ref,
                       rw_ref, rb_ref, gw_ref, gb_ref,
                       x_out_ref, u_out_ref):
    _, xh_new, u_new = _node_core(
        xh_ref, msga_ref, msgb_ref, deg_ref, b_ref, bt_ref, cntn_ref, u_ref,
        eg_ref, cnte_ref, wnx_ref, wnm_ref, wnu_ref, bn_ref,
        wge_ref, wgx_ref, wgu_ref, bg_ref)
    x_out_ref[...] = _sig(_mm(xh_new, rw_ref[...]) + rb_ref[...])
    u_out_ref[...] = _sig(_mm(u_new, gw_ref[...]) + gb_ref[...])


def _split_layer_weights(p, k, dx):
    mp = p['mp'][k]
    no = mp['node_w'].shape[1]
    wnx = mp['node_w'][:dx]
    wnm = mp['node_w'][dx:dx + 128]
    wge = mp['global_w'][:128]
    wgx = mp['global_w'][128:128 + no]
    if k == 0:
        wnu = jnp.zeros((32, no), jnp.float32)
        wgu = jnp.zeros((32, 32), jnp.float32)
    else:
        wnu = mp['node_w'][dx + 128:]
        wgu = mp['global_w'][128 + no:]
    return wnx, wnm, wnu, mp['node_b'][None], wge, wgx, wgu, mp['global_b'][None]


def _node_update(xh, msgp, deg2, B, BT, cntn2, u, eg, cnte2, p, k):
    N = xh.shape[0]
    dx = xh.shape[1]
    ws = _split_layer_weights(p, k, dx)
    no = ws[0].shape[1]
    args = (xh, msgp[0], msgp[1], deg2, B, BT, cntn2, u, eg,
            cnte2) + ws
    if k < 3:
        mpn = p['mp'][k + 1]
        wpx = mpn['edge_w'][:no]
        wpu = mpn['edge_w'][no + 128:]
        bp = mpn['edge_b'][None]
        return pl.pallas_call(
            _node_upd_kernel,
            out_shape=(jax.ShapeDtypeStruct((N, no), jnp.float32),
                       jax.ShapeDtypeStruct((32, 32), jnp.float32),
                       jax.ShapeDtypeStruct((N, 128), jnp.float32)),
        )(*args, wpx, wpu, bp)
    return pl.pallas_call(
        _node_final_kernel,
        out_shape=(jax.ShapeDtypeStruct((N, 2), jnp.float32),
                   jax.ShapeDtypeStruct((32, 5), jnp.float32)),
    )(*args, p['ro_node_w'], p['ro_node_b'][None],
      p['ro_global_w'], p['ro_global_b'][None])


_NW = 32
_CH = 128


def _sc_gather(P, idx2d):
    EP = idx2d.shape[0] * _CH
    per_w = EP // _NW
    k = per_w // _CH

    @functools.partial(
        pl.kernel,
        mesh=plsc.VectorSubcoreMesh(core_axis_name="c", subcore_axis_name="s"),
        out_type=jax.ShapeDtypeStruct((EP, 128), jnp.float32),
        scratch_types=[
            pltpu.VMEM((k, _CH), jnp.int32),
            pltpu.VMEM((_CH, 128), jnp.float32),
            pltpu.VMEM((_CH, 128), jnp.float32),
            pltpu.VMEM((_CH, 128), jnp.float32),
            pltpu.VMEM((_CH, 128), jnp.float32),
            pltpu.SemaphoreType.DMA,
            pltpu.SemaphoreType.DMA,
            pltpu.SemaphoreType.DMA,
            pltpu.SemaphoreType.DMA,
        ],
    )
    def gk(p_hbm, idx_hbm, out_hbm, idx_v,
           rows_v0, rows_v1, rows_v2, rows_v3, sem0, sem1, sem2, sem3):
        wid = lax.axis_index("s") * 2 + lax.axis_index("c")
        pltpu.sync_copy(idx_hbm.at[pl.ds(wid * k, k)], idx_v)
        rows = (rows_v0, rows_v1, rows_v2, rows_v3)
        sems = (sem0, sem1, sem2, sem3)
        base = wid * per_w
        pltpu.async_copy(p_hbm.at[idx_v.at[0]], rows[0], sems[0])
        pltpu.async_copy(p_hbm.at[idx_v.at[1]], rows[1], sems[1])

        def wait_any(b):
            pltpu.make_async_copy(p_hbm.at[idx_v.at[0]], rows[b],
                                  sems[b]).wait()

        def body(t, _):
            for b in range(4):
                j = 4 * t + b
                wait_any(b)
                pltpu.async_copy(rows[b],
                                 out_hbm.at[pl.ds(base + j * _CH, _CH)],
                                 sems[b])
                b2 = (b + 2) % 4
                if b >= 2:
                    wait_any(b2)
                else:
                    @pl.when(t > 0)
                    def _():
                        wait_any(b2)

                jn = jnp.minimum(j + 2, k - 1)
                pltpu.async_copy(p_hbm.at[idx_v.at[jn]], rows[b2], sems[b2])
            return 0

        lax.fori_loop(0, k // 4, body, 0)
        for b in range(2):
            wait_any(b)
        for b in range(2):
            wait_any((k - 2 + b) % 4)

    return gk(P, idx2d)


def _sc_scatter(e, dst2d, init, n_rows):
    EP = e.shape[0]
    per_w = EP // _NW
    k = per_w // _CH
    NA = ((n_rows + 8 + 631) // 632) * 632
    wr = 632
    nw_full = n_rows // wr
    tail = n_rows - nw_full * wr

    @functools.partial(
        pl.kernel,
        mesh=plsc.VectorSubcoreMesh(core_axis_name="c", subcore_axis_name="s"),
        out_type=jax.ShapeDtypeStruct((2, n_rows, 128), jnp.float32),
        scratch_types=[
            pltpu.VMEM((k, _CH), jnp.int32),
            pltpu.VMEM((_CH, 128), jnp.float32),
            pltpu.VMEM((_CH, 128), jnp.float32),
            pltpu.VMEM_SHARED((NA, 128), jnp.float32),
            pltpu.SemaphoreType.DMA,
            pltpu.SemaphoreType.DMA,
        ],
    )
    def sk(e_hbm, d_hbm, i_hbm, msg_hbm, di_v,
           rows_v0, rows_v1, acc_sh, sem0, sem1):
        c = lax.axis_index("c")
        s = lax.axis_index("s")
        wid = s * 2 + c
        rows = (rows_v0, rows_v1)
        sems = (sem0, sem1)
        pltpu.sync_copy(d_hbm.at[pl.ds(wid * k, k)], di_v)
        @pl.when(s < nw_full)
        def _():
            pltpu.sync_copy(i_hbm.at[c].at[pl.ds(s * wr, wr)],
                            acc_sh.at[pl.ds(s * wr, wr)])

        @pl.when(s == nw_full)
        def _():
            pltpu.sync_copy(i_hbm.at[c].at[pl.ds(nw_full * wr, tail)],
                            acc_sh.at[pl.ds(nw_full * wr, tail)])

        plsc.subcore_barrier()
        base = wid * per_w
        pltpu.async_copy(e_hbm.at[pl.ds(base, _CH)], rows[0], sems[0])
        pltpu.async_copy(e_hbm.at[pl.ds(base + _CH, _CH)], rows[1], sems[1])

        def body(t, _):
            for b in range(2):
                j = 2 * t + b
                pltpu.make_async_copy(e_hbm.at[pl.ds(0, _CH)], rows[b],
                                      sems[b]).wait()
                pltpu.sync_copy(rows[b], acc_sh.at[di_v.at[j]], add=True)
                jn = jnp.minimum(j + 2, k - 1)
                pltpu.async_copy(e_hbm.at[pl.ds(base + jn * _CH, _CH)],
                                 rows[b], sems[b])
            return 0

        lax.fori_loop(0, k // 2, body, 0)
        for b in range(2):
            pltpu.make_async_copy(e_hbm.at[pl.ds(0, _CH)], rows[b],
                                  sems[b]).wait()
        plsc.subcore_barrier()
        @pl.when(s < nw_full)
        def _():
            pltpu.sync_copy(acc_sh.at[pl.ds(s * wr, wr)],
                            msg_hbm.at[c].at[pl.ds(s * wr, wr)])

        @pl.when(s == nw_full)
        def _():
            pltpu.sync_copy(acc_sh.at[pl.ds(nw_full * wr, tail)],
                            msg_hbm.at[c].at[pl.ds(nw_full * wr, tail)])

    return sk(e, dst2d, init)


def kernel(aa, msa_feats, x, edge_index, edge_attr, secondary_structure,
           batch, params):
    p = params
    N = aa.shape[0]
    E = edge_index.shape[1]
    EP = ((E + 32767) // 32768) * 32768
    pad = EP - E

    src, dst = edge_index[0], edge_index[1]
    bsrc = batch[src]
    srcp = jnp.concatenate([src, jnp.zeros((pad,), src.dtype)])
    dstp = jnp.concatenate([dst, jnp.full((pad,), N, dst.dtype)])
    bsrcp = jnp.concatenate([bsrc, jnp.full((pad,), _NG, bsrc.dtype)])
    eap = jnp.concatenate([edge_attr, jnp.zeros((pad, 4), jnp.float32)])
    src2 = srcp[:, None]
    dst2 = dstp[:, None]
    src2d = srcp.reshape(EP // _CH, _CH).astype(jnp.int32)
    dst2d = dstp.reshape(EP // _CH, _CH).astype(jnp.int32)
    bsrc3 = bsrcp.reshape(EP // _BE, 1, _BE).astype(jnp.int32)
    zeros2 = jnp.zeros((2, N, 128), jnp.float32)
    EH = EP // 2
    h2d = EH // _CH
    h3d = EH // _BE

    deg2 = jnp.clip(jax.ops.segment_sum(jnp.ones((E,), jnp.float32), dst,
                                        num_segments=N), 1.0)[:, None]
    B = (batch[:, None] == jnp.arange(_NG)[None, :]).astype(jnp.float32)
    BT = B.T
    cntn2 = jnp.clip(jnp.sum(B, axis=0), 1.0)[:, None]

    nodes = jnp.concatenate([x, msa_feats], axis=1)
    xh, P = _node_encoder(nodes, aa[:, None].astype(jnp.int32),
                          secondary_structure[:, None].astype(jnp.int32), p)
    u = jnp.zeros((32, 32), jnp.float32)

    eh = [None] * (EP // EH)
    for k in range(4):
        eg40s = []
        msgp = zeros2
        for h in range(EP // EH):
            g = _sc_gather(P, src2d[h * h2d:(h + 1) * h2d])
            if k == 0:
                ehh, eg40 = _edge1(eap[h * EH:(h + 1) * EH],
                                   src2[h * EH:(h + 1) * EH],
                                   dst2[h * EH:(h + 1) * EH],
                                   bsrc3[h * h3d:(h + 1) * h3d], g, p)
            else:
                dx = xh.shape[1]
                wee = p['mp'][k]['edge_w'][dx:dx + 128]
                ehh, eg40 = _edgek(eh[h], bsrc3[h * h3d:(h + 1) * h3d],
                                   g, wee)
            eh[h] = ehh
            eg40s.append(eg40)
            msgp = _sc_scatter(ehh, dst2d[h * h2d:(h + 1) * h2d], msgp, N)
        eg40 = sum(eg40s)
        eg = eg40[:_NG, :128]
        cnte2 = jnp.clip(eg40[:_NG, 128:129], 1.0)
        out = _node_update(xh, msgp, deg2, B, BT, cntn2, u, eg,
                           cnte2, p, k)
        if k < 3:
            xh, u, P = out
        else:
            return out[0], out[1]
